# baseline (speedup 1.0000x reference)
import functools

import jax
import jax.numpy as jnp
from jax.experimental import pallas as pl
from jax.experimental.pallas import tpu as pltpu
from jax.experimental.pallas import tpu_sc as plsc

D_MODEL = 128
R = 8
C = 4


def _gather_rows(table, idx, B, S):
    mesh = plsc.VectorSubcoreMesh(core_axis_name="core",
                                  subcore_axis_name="subcore")

    @functools.partial(
        pl.kernel,
        out_type=jax.ShapeDtypeStruct((B, S, D_MODEL), table.dtype),
        mesh=mesh,
        scratch_types=[pltpu.SemaphoreType.DMA],
        compiler_params=pltpu.CompilerParams(use_tc_tiling_on_sc=True),
    )
    def gather_kernel(table_hbm, idx_hbm, out_hbm, sem):
        def body(i_vmem, o_vmem):
            copies = [
                pltpu.async_copy(table_hbm.at[i_vmem.at[r]],
                                 o_vmem.at[r], sem)
                for r in range(R)
            ]
            for c in copies:
                c.wait()

        pltpu.emit_pipeline(
            body,
            grid=(B // R,),
            in_specs=[pl.BlockSpec((R, S), index_map=lambda i: (i, 0))],
            out_specs=[pl.BlockSpec((R, S, D_MODEL),
                                    index_map=lambda i: (i, 0, 0))],
            core_axis_name=("core", "subcore"),
            dimension_semantics=(pltpu.PARALLEL,),
        )(idx_hbm, out_hbm)

    return gather_kernel(table, idx)


def kernel(x, table):
    B, S = x.shape
    idx = x.astype(jnp.int32)
    bc = B // C
    parts = [_gather_rows(table, idx[c * bc:(c + 1) * bc], bc, S)
             for c in range(C)]
    return jnp.concatenate(parts, axis=0)


# speedup vs baseline: 3.1786x; 3.1786x over previous
import functools

import jax
import jax.numpy as jnp
from jax.experimental import pallas as pl
from jax.experimental.pallas import tpu as pltpu
from jax.experimental.pallas import tpu_sc as plsc

D_MODEL = 128
W = 128
K = 2


def _gather_rows_t(table, idx3, S, B):
    nb = B // (K * W)
    mesh = plsc.VectorSubcoreMesh(core_axis_name="core",
                                  subcore_axis_name="subcore")

    @functools.partial(
        pl.kernel,
        out_type=jax.ShapeDtypeStruct((S, B, D_MODEL), table.dtype),
        mesh=mesh,
        scratch_types=[pltpu.SemaphoreType.DMA],
    )
    def gather_kernel(table_hbm, idx_hbm, out_hbm, sem):
        def body(i_vmem, o_vmem):
            copies = [
                pltpu.async_copy(table_hbm.at[i_vmem.at[0, k]],
                                 o_vmem.at[0, pl.ds(k * W, W)], sem)
                for k in range(K)
            ]
            for c in copies:
                c.wait()

        pltpu.emit_pipeline(
            body,
            grid=(S * nb,),
            in_specs=[pl.BlockSpec((1, K, W),
                                   index_map=lambda i: (i // nb, i % nb, 0))],
            out_specs=[pl.BlockSpec((1, K * W, D_MODEL),
                                    index_map=lambda i: (i // nb, i % nb, 0))],
            core_axis_name=("core", "subcore"),
            dimension_semantics=(pltpu.PARALLEL,),
        )(idx_hbm, out_hbm)

    return gather_kernel(table, idx3)


def kernel(x, table):
    B, S = x.shape
    idx3 = x.T.astype(jnp.int32).reshape(S, B // W, W)
    out_t = _gather_rows_t(table, idx3, S, B)
    return out_t.transpose(1, 0, 2)
